# baseline (speedup 1.0000x reference)
import functools

import jax
import jax.numpy as jnp
from jax.experimental import pallas as pl
from jax.experimental.pallas import tpu as pltpu

_MMDT = jnp.bfloat16
_EPS = 1e-6
_GROUPS = 32


def _gn_swish(x, grp_ref, gamma_ref, beta_ref, inv_n):
    s = jnp.sum(x, axis=0, keepdims=True)
    s2 = jnp.sum(x * x, axis=0, keepdims=True)
    gs = jnp.dot(s, grp_ref[...], preferred_element_type=jnp.float32)
    gs2 = jnp.dot(s2, grp_ref[...], preferred_element_type=jnp.float32)
    mean = gs * inv_n
    var = jnp.maximum(gs2 * inv_n - mean * mean, 0.0)
    a = jax.lax.rsqrt(var + _EPS) * gamma_ref[...]
    bb = beta_ref[...] - mean * a
    t = x * (0.5 * a) + 0.5 * bb
    return t + t * jnp.tanh(t)


def _fill_pad(pad_ref, y_hwc, H, W):
    C = pad_ref.shape[2]
    zrow = jnp.zeros((1, W + 2, C), pad_ref.dtype)
    zcol = jnp.zeros((H + 2, 1, C), pad_ref.dtype)
    pad_ref[0:1] = zrow
    pad_ref[H + 1:H + 2] = zrow
    pad_ref[:, 0:1, :] = zcol
    pad_ref[:, W + 1:W + 2, :] = zcol
    pad_ref[1:H + 1, 1:W + 1, :] = y_hwc.astype(pad_ref.dtype)


def _conv_from_pad(pad_ref, w_ref, H, W, C, Cout):
    acc = None
    for dx in range(3):
        xs = pad_ref[:, dx:dx + W, :].reshape((H + 2) * W, C)
        for dy in range(3):
            t = jnp.dot(xs[dy * W:dy * W + H * W], w_ref[3 * dy + dx],
                        preferred_element_type=jnp.float32)
            acc = t if acc is None else acc + t
    return acc


def _gn_swish_b(x, grp_ref, gamma_ref, beta_ref, inv_n):
    S, _, C = x.shape
    s = jnp.sum(x, axis=1, keepdims=True)
    s2 = jnp.sum(x * x, axis=1, keepdims=True)
    gs = jnp.dot(s.reshape(S, C), grp_ref[...],
                 preferred_element_type=jnp.float32).reshape(S, 1, C)
    gs2 = jnp.dot(s2.reshape(S, C), grp_ref[...],
                  preferred_element_type=jnp.float32).reshape(S, 1, C)
    mean = gs * inv_n
    var = jnp.maximum(gs2 * inv_n - mean * mean, 0.0)
    a = jax.lax.rsqrt(var + _EPS) * gamma_ref[...]
    bb = beta_ref[...] - mean * a
    t = x * (0.5 * a) + 0.5 * bb
    return t + t * jnp.tanh(t)


def _fill_pad_tall(pad_ref, y_smc, S, H, W):
    C = pad_ref.shape[2]
    pad_ref[...] = jnp.zeros(pad_ref.shape, pad_ref.dtype)
    Hp = H + 2
    for s in range(S):
        pad_ref[s * Hp + 1:s * Hp + 1 + H, 1:W + 1, :] = (
            y_smc[s].reshape(H, W, C).astype(pad_ref.dtype))


def _conv_tall(pad_ref, w_ref, S, H, W, C, Cout):
    Hp = H + 2
    Mtall = (S * Hp - 2) * W
    acc = None
    for dx in range(3):
        xs = pad_ref[:, dx:dx + W, :].reshape(S * Hp * W, C)
        for dy in range(3):
            t = jnp.dot(xs[dy * W:dy * W + Mtall], w_ref[3 * dy + dx],
                        preferred_element_type=jnp.float32)
            acc = t if acc is None else acc + t
    return jnp.stack([acc[s * Hp * W:s * Hp * W + H * W] for s in range(S)],
                     axis=0)


def _gnconv_body(x_ref, gamma_ref, beta_ref, grp_ref, w_ref, b_ref, o_ref,
                 pad_ref, *, H, W, C, Cout, inv_n, use_gn):
    xm = x_ref[0].reshape(H * W, C).astype(jnp.float32)
    y = _gn_swish(xm, grp_ref, gamma_ref, beta_ref, inv_n) if use_gn else xm
    _fill_pad(pad_ref, y.reshape(H, W, C), H, W)
    acc = _conv_from_pad(pad_ref, w_ref, H, W, C, Cout) + b_ref[...]
    o_ref[0] = acc.reshape(H, W, Cout)


def _conv_in_body_s(x_ref, w_ref, b_ref, o_ref, pad_ref, *, S, H, W, C,
                    Cout):
    xm = x_ref[...].reshape(S, H * W, C).astype(jnp.float32)
    _fill_pad_tall(pad_ref, xm, S, H, W)
    acc = _conv_tall(pad_ref, w_ref, S, H, W, C, Cout) + b_ref[...]
    o_ref[...] = acc.reshape(S, H, W, Cout)


def _res_body_s(x_ref, g0_ref, b0_ref, g1_ref, b1_ref, grp0_ref, grp1_ref,
                w0_ref, w1_ref, o_ref, pad0_ref, pad1_ref,
                *, S, H, W, Cin, Cout, inv0, inv1):
    xm = x_ref[...].reshape(S, H * W, Cin).astype(jnp.float32)
    y0 = _gn_swish_b(xm, grp0_ref, g0_ref, b0_ref, inv0)
    _fill_pad_tall(pad0_ref, y0, S, H, W)
    h = _conv_tall(pad0_ref, w0_ref, S, H, W, Cin, Cout)
    y1 = _gn_swish_b(h, grp1_ref, g1_ref, b1_ref, inv1)
    _fill_pad_tall(pad1_ref, y1, S, H, W)
    h2 = _conv_tall(pad1_ref, w1_ref, S, H, W, Cout, Cout)
    o_ref[...] = (h2 + xm).reshape(S, H, W, Cout)


def _res_body(x_ref, g0_ref, b0_ref, g1_ref, b1_ref, grp0_ref, grp1_ref,
              w0_ref, w1_ref, *rest, H, W, Cin, Cout, inv0, inv1, proj):
    if proj:
        wr_ref, o_ref, pad0_ref, pad1_ref = rest
    else:
        o_ref, pad0_ref, pad1_ref = rest
    xm = x_ref[0].reshape(H * W, Cin).astype(jnp.float32)
    y0 = _gn_swish(xm, grp0_ref, g0_ref, b0_ref, inv0)
    _fill_pad(pad0_ref, y0.reshape(H, W, Cin), H, W)
    h = _conv_from_pad(pad0_ref, w0_ref, H, W, Cin, Cout)
    y1 = _gn_swish(h, grp1_ref, g1_ref, b1_ref, inv1)
    _fill_pad(pad1_ref, y1.reshape(H, W, Cout), H, W)
    h2 = _conv_from_pad(pad1_ref, w1_ref, H, W, Cout, Cout)
    if proj:
        out = h2 + jnp.dot(h2.astype(_MMDT), wr_ref[...],
                           preferred_element_type=jnp.float32)
    else:
        out = h2 + xm
    o_ref[0] = out.reshape(H, W, Cout)


def _upconv_body(x_ref, w_ref, b_ref, o_ref, pad_ref, *, H, W, C):
    x = x_ref[0]
    xw = jnp.concatenate([x[:, :, None, :], x[:, :, None, :]],
                         axis=2).reshape(H, 2 * W, C)
    xh = jnp.concatenate([xw[:, None], xw[:, None]],
                         axis=1).reshape(2 * H, 2 * W, C)
    _fill_pad(pad_ref, xh, 2 * H, 2 * W)
    acc = _conv_from_pad(pad_ref, w_ref, 2 * H, 2 * W, C, C) + b_ref[...]
    o_ref[0] = acc.reshape(2 * H, 2 * W, C)


def _grp_matrix(C):
    gid = jnp.arange(C, dtype=jnp.int32) // (C // _GROUPS)
    return (gid[:, None] == gid[None, :]).astype(jnp.float32)


def _wmat3x3(w):
    Cout, Cin = w.shape[0], w.shape[1]
    return jnp.transpose(w, (2, 3, 1, 0)).reshape(9, Cin, Cout).astype(_MMDT)


def _bias_row(b, Cout):
    return (jnp.zeros((1, Cout), jnp.float32) if b is None
            else b.reshape(1, Cout).astype(jnp.float32))


_PAR = pltpu.CompilerParams(dimension_semantics=("parallel",))


def _conv_in_stacked(x, w, b, S):
    B, H, W, C = x.shape
    Cout = w.shape[0]
    body = functools.partial(_conv_in_body_s, S=S, H=H, W=W, C=C, Cout=Cout)
    return pl.pallas_call(
        body,
        out_shape=jax.ShapeDtypeStruct((B, H, W, Cout), jnp.float32),
        grid_spec=pltpu.PrefetchScalarGridSpec(
            num_scalar_prefetch=0,
            grid=(B // S,),
            in_specs=[
                pl.BlockSpec((S, H, W, C), lambda i: (i, 0, 0, 0)),
                pl.BlockSpec((9, C, Cout), lambda i: (0, 0, 0)),
                pl.BlockSpec((1, Cout), lambda i: (0, 0)),
            ],
            out_specs=pl.BlockSpec((S, H, W, Cout), lambda i: (i, 0, 0, 0)),
            scratch_shapes=[pltpu.VMEM((S * (H + 2), W + 2, C), _MMDT)],
        ),
        compiler_params=_PAR,
    )(x, _wmat3x3(w), _bias_row(b, Cout))


def _res_block_stacked(x, g0, b0, w0, g1, b1, w1, S):
    B, H, W, C = x.shape
    Cout = w0.shape[0]
    body = functools.partial(_res_body_s, S=S, H=H, W=W, Cin=C, Cout=Cout,
                             inv0=1.0 / float(H * W * (C // _GROUPS)),
                             inv1=1.0 / float(H * W * (Cout // _GROUPS)))
    return pl.pallas_call(
        body,
        out_shape=jax.ShapeDtypeStruct((B, H, W, Cout), jnp.float32),
        grid_spec=pltpu.PrefetchScalarGridSpec(
            num_scalar_prefetch=0,
            grid=(B // S,),
            in_specs=[
                pl.BlockSpec((S, H, W, C), lambda i: (i, 0, 0, 0)),
                pl.BlockSpec((1, C), lambda i: (0, 0)),
                pl.BlockSpec((1, C), lambda i: (0, 0)),
                pl.BlockSpec((1, Cout), lambda i: (0, 0)),
                pl.BlockSpec((1, Cout), lambda i: (0, 0)),
                pl.BlockSpec((C, C), lambda i: (0, 0)),
                pl.BlockSpec((Cout, Cout), lambda i: (0, 0)),
                pl.BlockSpec((9, C, Cout), lambda i: (0, 0, 0)),
                pl.BlockSpec((9, Cout, Cout), lambda i: (0, 0, 0)),
            ],
            out_specs=pl.BlockSpec((S, H, W, Cout), lambda i: (i, 0, 0, 0)),
            scratch_shapes=[pltpu.VMEM((S * (H + 2), W + 2, C), _MMDT),
                            pltpu.VMEM((S * (H + 2), W + 2, Cout), _MMDT)],
        ),
        compiler_params=_PAR,
    )(x,
      g0.reshape(1, C).astype(jnp.float32),
      b0.reshape(1, C).astype(jnp.float32),
      g1.reshape(1, Cout).astype(jnp.float32),
      b1.reshape(1, Cout).astype(jnp.float32),
      _grp_matrix(C), _grp_matrix(Cout), _wmat3x3(w0), _wmat3x3(w1))


def _gn_conv(x, gamma, beta, w, b, *, use_gn):
    B, H, W, C = x.shape
    Cout = w.shape[0]
    if use_gn:
        gam = gamma.reshape(1, C).astype(jnp.float32)
        bet = beta.reshape(1, C).astype(jnp.float32)
    else:
        gam = jnp.ones((1, C), jnp.float32)
        bet = jnp.zeros((1, C), jnp.float32)
    body = functools.partial(_gnconv_body, H=H, W=W, C=C, Cout=Cout,
                             inv_n=1.0 / float(H * W * (C // _GROUPS)),
                             use_gn=use_gn)
    return pl.pallas_call(
        body,
        out_shape=jax.ShapeDtypeStruct((B, H, W, Cout), jnp.float32),
        grid_spec=pltpu.PrefetchScalarGridSpec(
            num_scalar_prefetch=0,
            grid=(B,),
            in_specs=[
                pl.BlockSpec((1, H, W, C), lambda i: (i, 0, 0, 0)),
                pl.BlockSpec((1, C), lambda i: (0, 0)),
                pl.BlockSpec((1, C), lambda i: (0, 0)),
                pl.BlockSpec((C, C), lambda i: (0, 0)),
                pl.BlockSpec((9, C, Cout), lambda i: (0, 0, 0)),
                pl.BlockSpec((1, Cout), lambda i: (0, 0)),
            ],
            out_specs=pl.BlockSpec((1, H, W, Cout), lambda i: (i, 0, 0, 0)),
            scratch_shapes=[pltpu.VMEM((H + 2, W + 2, C), _MMDT)],
        ),
        compiler_params=_PAR,
    )(x, gam, bet, _grp_matrix(C), _wmat3x3(w), _bias_row(b, Cout))


def _res_block(x, g0, b0, w0, g1, b1, w1, wr):
    B, H, W, Cin = x.shape
    Cout = w0.shape[0]
    proj = wr is not None
    body = functools.partial(_res_body, H=H, W=W, Cin=Cin, Cout=Cout,
                             inv0=1.0 / float(H * W * (Cin // _GROUPS)),
                             inv1=1.0 / float(H * W * (Cout // _GROUPS)),
                             proj=proj)
    in_specs = [
        pl.BlockSpec((1, H, W, Cin), lambda i: (i, 0, 0, 0)),
        pl.BlockSpec((1, Cin), lambda i: (0, 0)),
        pl.BlockSpec((1, Cin), lambda i: (0, 0)),
        pl.BlockSpec((1, Cout), lambda i: (0, 0)),
        pl.BlockSpec((1, Cout), lambda i: (0, 0)),
        pl.BlockSpec((Cin, Cin), lambda i: (0, 0)),
        pl.BlockSpec((Cout, Cout), lambda i: (0, 0)),
        pl.BlockSpec((9, Cin, Cout), lambda i: (0, 0, 0)),
        pl.BlockSpec((9, Cout, Cout), lambda i: (0, 0, 0)),
    ]
    args = [x,
            g0.reshape(1, Cin).astype(jnp.float32),
            b0.reshape(1, Cin).astype(jnp.float32),
            g1.reshape(1, Cout).astype(jnp.float32),
            b1.reshape(1, Cout).astype(jnp.float32),
            _grp_matrix(Cin), _grp_matrix(Cout), _wmat3x3(w0), _wmat3x3(w1)]
    if proj:
        in_specs.append(pl.BlockSpec((Cout, Cout), lambda i: (0, 0)))
        args.append(jnp.transpose(wr.reshape(Cout, Cout)).astype(_MMDT))
    return pl.pallas_call(
        body,
        out_shape=jax.ShapeDtypeStruct((B, H, W, Cout), jnp.float32),
        grid_spec=pltpu.PrefetchScalarGridSpec(
            num_scalar_prefetch=0,
            grid=(B,),
            in_specs=in_specs,
            out_specs=pl.BlockSpec((1, H, W, Cout), lambda i: (i, 0, 0, 0)),
            scratch_shapes=[pltpu.VMEM((H + 2, W + 2, Cin), _MMDT),
                            pltpu.VMEM((H + 2, W + 2, Cout), _MMDT)],
        ),
        compiler_params=_PAR,
    )(*args)


def _up_conv(x, w, b):
    B, H, W, C = x.shape
    Cout = w.shape[0]
    body = functools.partial(_upconv_body, H=H, W=W, C=C)
    return pl.pallas_call(
        body,
        out_shape=jax.ShapeDtypeStruct((B, 2 * H, 2 * W, Cout), jnp.float32),
        grid_spec=pltpu.PrefetchScalarGridSpec(
            num_scalar_prefetch=0,
            grid=(B,),
            in_specs=[
                pl.BlockSpec((1, H, W, C), lambda i: (i, 0, 0, 0)),
                pl.BlockSpec((9, C, Cout), lambda i: (0, 0, 0)),
                pl.BlockSpec((1, Cout), lambda i: (0, 0)),
            ],
            out_specs=pl.BlockSpec((1, 2 * H, 2 * W, Cout),
                                   lambda i: (i, 0, 0, 0)),
            scratch_shapes=[pltpu.VMEM((2 * H + 2, 2 * W + 2, C), _MMDT)],
        ),
        compiler_params=_PAR,
    )(x, _wmat3x3(w), _bias_row(b, Cout))


def kernel(x, p00, p01, p02, p03, p04, p05, p06, p07, p08, p09, p10, p11,
           p12, p13, p14, p15, p16, p17, p18, p19, p20, p21, p22, p23, p24,
           p25, p26):
    B = x.shape[0]
    S = 4 if B % 4 == 0 else (2 if B % 2 == 0 else 1)
    h = jnp.transpose(x, (0, 2, 3, 1)).astype(jnp.float32)
    h = _conv_in_stacked(h, p01, p00, S)
    h = _res_block_stacked(h, p09, p08, p06, p11, p10, p07, S)
    h = _res_block_stacked(h, p15, p14, p12, p17, p16, p13, S)
    h = _up_conv(h, p19, p18)
    h = _res_block(h, p24, p23, p20, p26, p25, p21, p22)
    h = _gn_conv(h, p05, p04, p03, p02, use_gn=True)
    return jnp.transpose(h, (0, 3, 1, 2))


# speedup vs baseline: 1.0244x; 1.0244x over previous
import functools

import jax
import jax.numpy as jnp
from jax.experimental import pallas as pl
from jax.experimental.pallas import tpu as pltpu

_MMDT = jnp.bfloat16
_EPS = 1e-6
_GROUPS = 32


def _gn_swish(x, grp_ref, gamma_ref, beta_ref, inv_n):
    s = jnp.sum(x, axis=0, keepdims=True)
    s2 = jnp.sum(x * x, axis=0, keepdims=True)
    gs = jnp.dot(s, grp_ref[...], preferred_element_type=jnp.float32)
    gs2 = jnp.dot(s2, grp_ref[...], preferred_element_type=jnp.float32)
    mean = gs * inv_n
    var = jnp.maximum(gs2 * inv_n - mean * mean, 0.0)
    a = jax.lax.rsqrt(var + _EPS) * gamma_ref[...]
    bb = beta_ref[...] - mean * a
    t = x * (0.5 * a) + 0.5 * bb
    return t + t * jnp.tanh(t)


def _fill_pad(pad_ref, y_hwc, H, W):
    C = pad_ref.shape[2]
    zrow = jnp.zeros((1, W + 2, C), pad_ref.dtype)
    zcol = jnp.zeros((H + 2, 1, C), pad_ref.dtype)
    pad_ref[0:1] = zrow
    pad_ref[H + 1:H + 2] = zrow
    pad_ref[:, 0:1, :] = zcol
    pad_ref[:, W + 1:W + 2, :] = zcol
    pad_ref[1:H + 1, 1:W + 1, :] = y_hwc.astype(pad_ref.dtype)


def _conv_from_pad(pad_ref, w_ref, H, W, C, Cout):
    acc = None
    for dx in range(3):
        xs = pad_ref[:, dx:dx + W, :].reshape((H + 2) * W, C)
        for dy in range(3):
            t = jnp.dot(xs[dy * W:dy * W + H * W], w_ref[3 * dy + dx],
                        preferred_element_type=jnp.float32)
            acc = t if acc is None else acc + t
    return acc


def _gnconv_body(x_ref, gamma_ref, beta_ref, grp_ref, w_ref, b_ref, o_ref,
                 pad_ref, *, H, W, C, Cout, inv_n, use_gn):
    xm = x_ref[0].reshape(H * W, C).astype(jnp.float32)
    y = _gn_swish(xm, grp_ref, gamma_ref, beta_ref, inv_n) if use_gn else xm
    _fill_pad(pad_ref, y.reshape(H, W, C), H, W)
    acc = _conv_from_pad(pad_ref, w_ref, H, W, C, Cout) + b_ref[...]
    o_ref[0] = acc.reshape(H, W, Cout)


def _res_body(x_ref, g0_ref, b0_ref, g1_ref, b1_ref, grp0_ref, grp1_ref,
              w0_ref, w1_ref, *rest, H, W, Cin, Cout, inv0, inv1, proj):
    if proj:
        wr_ref, o_ref, pad0_ref, pad1_ref = rest
    else:
        o_ref, pad0_ref, pad1_ref = rest
    xm = x_ref[0].reshape(H * W, Cin).astype(jnp.float32)
    y0 = _gn_swish(xm, grp0_ref, g0_ref, b0_ref, inv0)
    _fill_pad(pad0_ref, y0.reshape(H, W, Cin), H, W)
    h = _conv_from_pad(pad0_ref, w0_ref, H, W, Cin, Cout)
    y1 = _gn_swish(h, grp1_ref, g1_ref, b1_ref, inv1)
    _fill_pad(pad1_ref, y1.reshape(H, W, Cout), H, W)
    h2 = _conv_from_pad(pad1_ref, w1_ref, H, W, Cout, Cout)
    if proj:
        out = h2 + jnp.dot(h2.astype(_MMDT), wr_ref[...],
                           preferred_element_type=jnp.float32)
    else:
        out = h2 + xm
    o_ref[0] = out.reshape(H, W, Cout)


def _upconv_body(x_ref, w_ref, b_ref, o_ref, pad_ref, *, H, W, C):
    x = x_ref[0]
    xw = jnp.concatenate([x[:, :, None, :], x[:, :, None, :]],
                         axis=2).reshape(H, 2 * W, C)
    xh = jnp.concatenate([xw[:, None], xw[:, None]],
                         axis=1).reshape(2 * H, 2 * W, C)
    _fill_pad(pad_ref, xh, 2 * H, 2 * W)
    acc = _conv_from_pad(pad_ref, w_ref, 2 * H, 2 * W, C, C) + b_ref[...]
    o_ref[0] = acc.reshape(2 * H, 2 * W, C)


def _grp_matrix(C):
    gid = jnp.arange(C, dtype=jnp.int32) // (C // _GROUPS)
    return (gid[:, None] == gid[None, :]).astype(jnp.float32)


def _wmat3x3(w):
    Cout, Cin = w.shape[0], w.shape[1]
    return jnp.transpose(w, (2, 3, 1, 0)).reshape(9, Cin, Cout).astype(_MMDT)


def _bias_row(b, Cout):
    return (jnp.zeros((1, Cout), jnp.float32) if b is None
            else b.reshape(1, Cout).astype(jnp.float32))


_PAR = pltpu.CompilerParams(dimension_semantics=("parallel",))


def _gn_conv(x, gamma, beta, w, b, *, use_gn):
    B, H, W, C = x.shape
    Cout = w.shape[0]
    if use_gn:
        gam = gamma.reshape(1, C).astype(jnp.float32)
        bet = beta.reshape(1, C).astype(jnp.float32)
    else:
        gam = jnp.ones((1, C), jnp.float32)
        bet = jnp.zeros((1, C), jnp.float32)
    body = functools.partial(_gnconv_body, H=H, W=W, C=C, Cout=Cout,
                             inv_n=1.0 / float(H * W * (C // _GROUPS)),
                             use_gn=use_gn)
    return pl.pallas_call(
        body,
        out_shape=jax.ShapeDtypeStruct((B, H, W, Cout), jnp.float32),
        grid_spec=pltpu.PrefetchScalarGridSpec(
            num_scalar_prefetch=0,
            grid=(B,),
            in_specs=[
                pl.BlockSpec((1, H, W, C), lambda i: (i, 0, 0, 0)),
                pl.BlockSpec((1, C), lambda i: (0, 0)),
                pl.BlockSpec((1, C), lambda i: (0, 0)),
                pl.BlockSpec((C, C), lambda i: (0, 0)),
                pl.BlockSpec((9, C, Cout), lambda i: (0, 0, 0)),
                pl.BlockSpec((1, Cout), lambda i: (0, 0)),
            ],
            out_specs=pl.BlockSpec((1, H, W, Cout), lambda i: (i, 0, 0, 0)),
            scratch_shapes=[pltpu.VMEM((H + 2, W + 2, C), _MMDT)],
        ),
        compiler_params=_PAR,
    )(x, gam, bet, _grp_matrix(C), _wmat3x3(w), _bias_row(b, Cout))


def _res_block(x, g0, b0, w0, g1, b1, w1, wr):
    B, H, W, Cin = x.shape
    Cout = w0.shape[0]
    proj = wr is not None
    body = functools.partial(_res_body, H=H, W=W, Cin=Cin, Cout=Cout,
                             inv0=1.0 / float(H * W * (Cin // _GROUPS)),
                             inv1=1.0 / float(H * W * (Cout // _GROUPS)),
                             proj=proj)
    in_specs = [
        pl.BlockSpec((1, H, W, Cin), lambda i: (i, 0, 0, 0)),
        pl.BlockSpec((1, Cin), lambda i: (0, 0)),
        pl.BlockSpec((1, Cin), lambda i: (0, 0)),
        pl.BlockSpec((1, Cout), lambda i: (0, 0)),
        pl.BlockSpec((1, Cout), lambda i: (0, 0)),
        pl.BlockSpec((Cin, Cin), lambda i: (0, 0)),
        pl.BlockSpec((Cout, Cout), lambda i: (0, 0)),
        pl.BlockSpec((9, Cin, Cout), lambda i: (0, 0, 0)),
        pl.BlockSpec((9, Cout, Cout), lambda i: (0, 0, 0)),
    ]
    args = [x,
            g0.reshape(1, Cin).astype(jnp.float32),
            b0.reshape(1, Cin).astype(jnp.float32),
            g1.reshape(1, Cout).astype(jnp.float32),
            b1.reshape(1, Cout).astype(jnp.float32),
            _grp_matrix(Cin), _grp_matrix(Cout), _wmat3x3(w0), _wmat3x3(w1)]
    if proj:
        in_specs.append(pl.BlockSpec((Cout, Cout), lambda i: (0, 0)))
        args.append(jnp.transpose(wr.reshape(Cout, Cout)).astype(_MMDT))
    return pl.pallas_call(
        body,
        out_shape=jax.ShapeDtypeStruct((B, H, W, Cout), jnp.float32),
        grid_spec=pltpu.PrefetchScalarGridSpec(
            num_scalar_prefetch=0,
            grid=(B,),
            in_specs=in_specs,
            out_specs=pl.BlockSpec((1, H, W, Cout), lambda i: (i, 0, 0, 0)),
            scratch_shapes=[pltpu.VMEM((H + 2, W + 2, Cin), _MMDT),
                            pltpu.VMEM((H + 2, W + 2, Cout), _MMDT)],
        ),
        compiler_params=_PAR,
    )(*args)


def _up_conv(x, w, b):
    B, H, W, C = x.shape
    Cout = w.shape[0]
    body = functools.partial(_upconv_body, H=H, W=W, C=C)
    return pl.pallas_call(
        body,
        out_shape=jax.ShapeDtypeStruct((B, 2 * H, 2 * W, Cout), jnp.float32),
        grid_spec=pltpu.PrefetchScalarGridSpec(
            num_scalar_prefetch=0,
            grid=(B,),
            in_specs=[
                pl.BlockSpec((1, H, W, C), lambda i: (i, 0, 0, 0)),
                pl.BlockSpec((9, C, Cout), lambda i: (0, 0, 0)),
                pl.BlockSpec((1, Cout), lambda i: (0, 0)),
            ],
            out_specs=pl.BlockSpec((1, 2 * H, 2 * W, Cout),
                                   lambda i: (i, 0, 0, 0)),
            scratch_shapes=[pltpu.VMEM((2 * H + 2, 2 * W + 2, C), _MMDT)],
        ),
        compiler_params=_PAR,
    )(x, _wmat3x3(w), _bias_row(b, Cout))


def kernel(x, p00, p01, p02, p03, p04, p05, p06, p07, p08, p09, p10, p11,
           p12, p13, p14, p15, p16, p17, p18, p19, p20, p21, p22, p23, p24,
           p25, p26):
    h = jnp.transpose(x, (0, 2, 3, 1)).astype(jnp.float32)
    h = _gn_conv(h, None, None, p01, p00, use_gn=False)
    h = _res_block(h, p09, p08, p06, p11, p10, p07, None)
    h = _res_block(h, p15, p14, p12, p17, p16, p13, None)
    h = _up_conv(h, p19, p18)
    h = _res_block(h, p24, p23, p20, p26, p25, p21, p22)
    h = _gn_conv(h, p05, p04, p03, p02, use_gn=True)
    return jnp.transpose(h, (0, 3, 1, 2))
